# baseline (speedup 1.0000x reference)
import jax
import jax.numpy as jnp
from jax import lax
from jax.experimental import pallas as pl
from jax.experimental.pallas import tpu as pltpu
from jax.experimental.pallas import tpu_sc as plsc

N = 16384
DIMS = (16, 16, 16, 32, 32, 32, 32)
COLS = (0, 16, 32, 48, 80, 112, 144)
TOTAL = 176

NC, NS = 2, 16
NW = NC * NS
BPW = N // NW
CHUNK = 128
NCH = BPW // CHUNK

_mesh = plsc.VectorSubcoreMesh(core_axis_name="c", subcore_axis_name="s")


def _body(idx_hbm, wh, wl, wr, wlon, wlat, out, idx_v, r16, r32, gsems, wsem):
    tables = (wh, wl, wr, wlon, wlat, wlon, wlat)
    wid = lax.axis_index("s") * NC + lax.axis_index("c")
    base = wid * BPW

    def rbuf(j, c):
        return r16.at[j, c] if j < 3 else r32.at[j - 3, c]

    gathers = []
    for j in range(7):
        pltpu.sync_copy(idx_hbm.at[j, wid], idx_v.at[j])
        for c in range(NCH):
            gathers.append(
                pltpu.async_copy(tables[j].at[idx_v.at[j, c]], rbuf(j, c),
                                 gsems.at[j]))

    writes = []
    for j in range(7):
        for c in range(NCH):
            gathers[j * NCH + c].wait()
        for c in range(NCH):
            writes.append(
                pltpu.async_copy(
                    rbuf(j, c),
                    out.at[pl.ds(base + c * CHUNK, CHUNK),
                           pl.ds(COLS[j], DIMS[j])],
                    wsem))
    for w in writes:
        w.wait()


_emb = pl.kernel(
    _body,
    out_type=jax.ShapeDtypeStruct((N, TOTAL), jnp.float32),
    mesh=_mesh,
    compiler_params=pltpu.CompilerParams(use_tc_tiling_on_sc=False),
    scratch_types=[
        pltpu.VMEM((7, NCH, CHUNK), jnp.int32),
        pltpu.VMEM((3, NCH, CHUNK, 16), jnp.float32),
        pltpu.VMEM((4, NCH, CHUNK, 32), jnp.float32),
        pltpu.SemaphoreType.DMA((7,)),
        pltpu.SemaphoreType.DMA,
    ],
)


def kernel(inputs, W_highway, W_length, W_radian, W_lon, W_lat):
    idx = jnp.transpose(inputs[:, 2:9]).reshape(7, NW, NCH, CHUNK)
    return _emb(idx, W_highway, W_length, W_radian, W_lon, W_lat)
